# baseline (speedup 1.0000x reference)
import functools

import jax
import jax.numpy as jnp
from jax import lax
from jax.experimental import pallas as pl
from jax.experimental.pallas import tpu as pltpu
from jax.experimental.pallas import tpu_sc as plsc

_D = 32
_R = 16384
_K = 20
_KP = 128
_B = _R * _K

_info = plsc.get_sparse_core_info()
_NC = _info.num_cores
_NS = _info.num_subcores
_NW = _NC * _NS
_RPW = _R // _NW
_CR = 64
_NCHUNK = _RPW // _CR
_CB = _CR * _K
_KG = 24

_mesh = plsc.VectorSubcoreMesh(core_axis_name="c", subcore_axis_name="s")


@functools.partial(
    pl.kernel,
    mesh=_mesh,
    out_type=jax.ShapeDtypeStruct((_B, _D), jnp.float32),
    scratch_types=[
        [pltpu.VMEM((_CR, _KP), jnp.int32) for _ in range(2)],
        [pltpu.VMEM((_CR * _KG, _D), jnp.float32) for _ in range(2)],
        [pltpu.SemaphoreType.DMA for _ in range(2)],
        [pltpu.SemaphoreType.DMA for _ in range(2)],
        [pltpu.SemaphoreType.DMA for _ in range(2)],
    ],
    compiler_params=pltpu.CompilerParams(use_tc_tiling_on_sc=False),
)
def _gather_kernel(table_hbm, idx_hbm, out_hbm, idx_v, obuf, isem, gsem, wsem):
    wid = lax.axis_index("s") * _NC + lax.axis_index("c")
    rbase = wid * _RPW
    obase = wid * _RPW * _K

    def load_idx(c, b):
        pltpu.async_copy(
            idx_hbm.at[pl.ds(rbase + c * _CR, _CR), :], idx_v[b], isem[b]
        )

    def wait_idx(c, b):
        pltpu.make_async_copy(
            idx_hbm.at[pl.ds(rbase + c * _CR, _CR), :], idx_v[b], isem[b]
        ).wait()

    def issue_chunk(c, b):
        def row(j, carry):
            pltpu.async_copy(
                table_hbm.at[idx_v[b].at[j, pl.ds(0, _KG)]],
                obuf[b].at[pl.ds(j * _KG, _KG), :],
                gsem[b],
            )
            return carry

        lax.fori_loop(0, _CR, row, 0)

    def drain_chunk(b):
        pltpu.make_async_copy(
            table_hbm.at[pl.ds(0, _CR * _KG), :], obuf[b], gsem[b]
        ).wait()

    def write_chunk(c, b):
        def row(j, carry):
            pltpu.async_copy(
                obuf[b].at[pl.ds(j * _KG, _K), :],
                out_hbm.at[pl.ds(obase + c * _CB + j * _K, _K), :],
                wsem[b],
            )
            return carry

        lax.fori_loop(0, _CR, row, 0)

    def wait_write(c, b):
        pltpu.make_async_copy(
            obuf[b].at[pl.ds(0, _CB), :],
            out_hbm.at[pl.ds(obase + c * _CB, _CB), :],
            wsem[b],
        ).wait()

    load_idx(0, 0)
    load_idx(1, 1)
    wait_idx(0, 0)
    issue_chunk(0, 0)
    for c in range(_NCHUNK):
        b = c % 2
        drain_chunk(b)
        if c + 1 < _NCHUNK:
            wait_idx(c + 1, 1 - b)
            if c >= 1:
                wait_write(c - 1, 1 - b)
            issue_chunk(c + 1, 1 - b)
            if c + 2 < _NCHUNK:
                load_idx(c + 2, b)
        write_chunk(c, b)
    wait_write(_NCHUNK - 2, _NCHUNK % 2)
    wait_write(_NCHUNK - 1, 1 - _NCHUNK % 2)


def kernel(arg1_1, arg223_1):
    idx = jnp.pad(arg223_1.astype(jnp.int32), ((0, 0), (0, _KP - _K)))
    return _gather_kernel(arg1_1, idx)


# speedup vs baseline: 1.2883x; 1.2883x over previous
import functools

import jax
import jax.numpy as jnp
from jax import lax
from jax.experimental import pallas as pl
from jax.experimental.pallas import tpu as pltpu
from jax.experimental.pallas import tpu_sc as plsc

_D = 32
_R = 16384
_K = 20
_KP = 128
_B = _R * _K
_V = 1000000
_Q = _V // 4

_info = plsc.get_sparse_core_info()
_NC = _info.num_cores
_NS = _info.num_subcores
_NW = _NC * _NS
_RPW = _R // _NW
_BPW = _RPW * _K
_WIN = 16
_NWIN = _RPW // _WIN
_LPW = _WIN * _K
_CQ = 256
_NCHUNK = _BPW // _CQ
_NPAIR = _NCHUNK // 2

_mesh = plsc.VectorSubcoreMesh(core_axis_name="c", subcore_axis_name="s")


@functools.partial(
    pl.kernel,
    mesh=_mesh,
    out_type=jax.ShapeDtypeStruct((_B, _D), jnp.float32),
    scratch_types=[
        pltpu.VMEM((_WIN, _KP), jnp.int32),
        pltpu.VMEM((_BPW,), jnp.int32),
        pltpu.VMEM((_BPW,), jnp.int32),
        [pltpu.VMEM((_CQ, _KP), jnp.float32) for _ in range(2)],
        [pltpu.VMEM((_CQ, _D), jnp.float32) for _ in range(2)],
        [pltpu.SemaphoreType.DMA for _ in range(2)],
        [pltpu.SemaphoreType.DMA for _ in range(2)],
    ],
    compiler_params=pltpu.CompilerParams(
        use_tc_tiling_on_sc=False, needs_layout_passes=False
    ),
)
def _gather_kernel(tq_hbm, idx_hbm, out_hbm, iwin, iq_v, bo_v, quad, obuf,
                   gsem, wsem):
    wid = lax.axis_index("s") * _NC + lax.axis_index("c")
    rbase = wid * _RPW
    obase = wid * _BPW

    lanes = lax.iota(jnp.int32, 16)

    def win_body(j, carry):
        pltpu.sync_copy(idx_hbm.at[pl.ds(rbase + j * _WIN, _WIN), :], iwin)

        def grp_body(g, carry2):
            t = lanes + g * 16
            rv = t // _K
            cv = t - rv * _K
            v = plsc.load_gather(iwin, [rv, cv])
            iq_v[pl.ds(j * _LPW + g * 16, 16)] = v >> 2
            bo_v[pl.ds(j * _LPW + g * 16, 16)] = (v & 3) << 5
            return carry2

        lax.fori_loop(0, _LPW // 16, grp_body, 0)
        return carry

    lax.fori_loop(0, _NWIN, win_body, 0)

    def fire(c, b):
        pltpu.async_copy(
            tq_hbm.at[iq_v.at[pl.ds(c * _CQ, _CQ)]], quad[b], gsem[b]
        )

    def drain(c, b):
        pltpu.make_async_copy(
            tq_hbm.at[iq_v.at[pl.ds(c * _CQ, _CQ)]], quad[b], gsem[b]
        ).wait()

    def extract(c, b):
        def grp_body(g, carry):
            rv = lanes + g * 16
            bov = bo_v[pl.ds(c * _CQ + g * 16, 16)]
            for col in range(_D):
                val = plsc.load_gather(quad[b], [rv, bov + col])
                plsc.store_scatter(obuf[b], [rv, lanes * 0 + col], val)
            return carry

        lax.fori_loop(0, _CQ // 16, grp_body, 0)

    def write(c, b):
        pltpu.async_copy(
            obuf[b], out_hbm.at[pl.ds(obase + c * _CQ, _CQ), :], wsem[b]
        )

    def wait_write(c, b):
        pltpu.make_async_copy(
            obuf[b], out_hbm.at[pl.ds(obase + c * _CQ, _CQ), :], wsem[b]
        ).wait()

    fire(0, 0)
    fire(1, 1)

    def pair_body(p, carry):
        for b in range(2):
            c = 2 * p + b
            drain(c, b)

            @pl.when(p >= 1)
            def _():
                wait_write(c - 2, b)

            extract(c, b)

            @pl.when(p < _NPAIR - 1)
            def _():
                fire(c + 2, b)

            write(c, b)
        return carry

    lax.fori_loop(0, _NPAIR, pair_body, 0)
    wait_write(_NCHUNK - 2, 0)
    wait_write(_NCHUNK - 1, 1)


def kernel(arg1_1, arg223_1):
    tq = arg1_1.reshape(_Q, 4 * _D)
    idx = jnp.pad(arg223_1.astype(jnp.int32), ((0, 0), (0, _KP - _K)))
    return _gather_kernel(tq, idx)


# speedup vs baseline: 1.9831x; 1.5394x over previous
import functools

import jax
import jax.numpy as jnp
from jax import lax
from jax.experimental import pallas as pl
from jax.experimental.pallas import tpu as pltpu
from jax.experimental.pallas import tpu_sc as plsc

_D = 32
_B = 16384 * 20

_info = plsc.get_sparse_core_info()
_NC = _info.num_cores
_NS = _info.num_subcores
_NW = _NC * _NS
_BPW = _B // _NW
_C = 1024
_NCHUNK = _BPW // _C
_NBUF = 3

_mesh = plsc.VectorSubcoreMesh(core_axis_name="c", subcore_axis_name="s")


@functools.partial(
    pl.kernel,
    mesh=_mesh,
    out_type=jax.ShapeDtypeStruct((_B, _D), jnp.float32),
    scratch_types=[
        pltpu.VMEM((_BPW,), jnp.int32),
        [pltpu.VMEM((_C, _D), jnp.float32) for _ in range(_NBUF)],
        [pltpu.SemaphoreType.DMA for _ in range(_NBUF)],
        [pltpu.SemaphoreType.DMA for _ in range(_NBUF)],
    ],
    compiler_params=pltpu.CompilerParams(use_tc_tiling_on_sc=False),
)
def _gather_kernel(table_hbm, idx_hbm, out_hbm, idx_v, rows, gsem, wsem):
    wid = lax.axis_index("s") * _NC + lax.axis_index("c")
    base = wid * _BPW

    pltpu.sync_copy(idx_hbm.at[pl.ds(base, _BPW)], idx_v)

    def fire_gather(i, b):
        pltpu.async_copy(
            table_hbm.at[idx_v.at[pl.ds(i * _C, _C)]], rows[b], gsem[b]
        )

    for i in range(_NBUF):
        fire_gather(i, i)

    for i in range(_NCHUNK):
        b = i % _NBUF
        pltpu.make_async_copy(
            table_hbm.at[idx_v.at[pl.ds(i * _C, _C)]], rows[b], gsem[b]
        ).wait()
        wcopy = pltpu.make_async_copy(
            rows[b], out_hbm.at[pl.ds(base + i * _C, _C)], wsem[b]
        )
        wcopy.start()
        ni = i + _NBUF
        if ni < _NCHUNK:
            wcopy.wait()
            fire_gather(ni, b)

    for j in range(max(0, _NCHUNK - _NBUF), _NCHUNK):
        bb = j % _NBUF
        pltpu.make_async_copy(
            rows[bb], out_hbm.at[pl.ds(base + j * _C, _C)], wsem[bb]
        ).wait()


def kernel(arg1_1, arg223_1):
    idx = arg223_1.reshape(-1).astype(jnp.int32)
    return _gather_kernel(arg1_1, idx)
